# baseline (speedup 1.0000x reference)
import functools

import jax
import jax.numpy as jnp
from jax import lax
from jax.experimental import pallas as pl
from jax.experimental.pallas import tpu as pltpu
from jax.experimental.pallas import tpu_sc as plsc

_NUM_EMB = 8192
_DIM = 32
_N_TOK = 8192
_TOK_TILE = 1024
_K_TILE = 512
_N_K_TILES = _NUM_EMB // _K_TILE
_K_CHUNK = 2048
_N_CHUNKS = _NUM_EMB // _K_CHUNK
_TILES_PER_CHUNK = _K_CHUNK // _K_TILE

_NC = 2
_NS = 16
_NW = _NC * _NS
_BPW = _N_TOK // _NW
_GCHUNK = 128


def _argmin_body(zt_ref, zsq_ref, cb_ref, idx_ref):
    zt = zt_ref[...]
    zsq = zsq_ref[...].reshape(_TOK_TILE)
    ztb = zt.astype(jnp.bfloat16)

    def tile_step(k, carry):
        rv, rt = carry
        cbk = cb_ref[pl.ds(k * _K_TILE, _K_TILE), :]
        csq = jnp.sum(cbk * cbk, axis=1)
        mm = lax.dot_general(ztb, cbk, (((1,), (1,)), ((), ())),
                             preferred_element_type=jnp.float32)
        d = (zsq[:, None] + csq[None, :]) - 2.0 * mm
        upd = d < rv
        return jnp.where(upd, d, rv), jnp.where(upd, k, rt)

    def chunk_step(c, carry):
        acc_val, acc_idx = carry
        init = (jnp.full((_TOK_TILE, _K_TILE), jnp.inf, jnp.float32),
                jnp.zeros((_TOK_TILE, _K_TILE), jnp.int32))
        rv, rt = lax.fori_loop(c * _TILES_PER_CHUNK, (c + 1) * _TILES_PER_CHUNK,
                               tile_step, init)
        v = jnp.min(rv, axis=1)
        lane = lax.broadcasted_iota(jnp.int32, (_TOK_TILE, _K_TILE), 1)
        gidx = rt * _K_TILE + lane
        i = jnp.min(jnp.where(rv == v[:, None], gidx, jnp.int32(2**30)), axis=1)
        acc_r = acc_val.astype(jnp.bfloat16).astype(jnp.float32)
        better = v < acc_r
        return jnp.where(better, v, acc_r), jnp.where(better, i, acc_idx)

    init = (jnp.full((_TOK_TILE,), jnp.inf, jnp.float32),
            jnp.zeros((_TOK_TILE,), jnp.int32))
    _, run_idx = lax.fori_loop(0, _N_CHUNKS, chunk_step, init)
    idx_ref[...] = run_idx.reshape(1, 1, _TOK_TILE)


def _compute_indices(z_flat, zsq3, codebook):
    return pl.pallas_call(
        _argmin_body,
        grid=(_N_TOK // _TOK_TILE,),
        in_specs=[
            pl.BlockSpec((_TOK_TILE, _DIM), lambda b: (b, 0)),
            pl.BlockSpec((1, 1, _TOK_TILE), lambda b: (b, 0, 0)),
            pl.BlockSpec((_NUM_EMB, _DIM), lambda b: (0, 0)),
        ],
        out_specs=pl.BlockSpec((1, 1, _TOK_TILE), lambda b: (b, 0, 0)),
        out_shape=jax.ShapeDtypeStruct((_N_TOK // _TOK_TILE, 1, _TOK_TILE),
                                       jnp.int32),
    )(z_flat, zsq3, codebook)


def _gather_body(cb_hbm, idx_hbm, out_hbm, idx_v, rows_v, sem):
    wid = lax.axis_index("s") * _NC + lax.axis_index("c")
    nrow = _BPW // _GCHUNK
    pltpu.sync_copy(idx_hbm.at[pl.ds(wid * nrow, nrow)], idx_v)
    for j in range(nrow):
        pltpu.async_copy(cb_hbm.at[idx_v.at[j]],
                         rows_v.at[pl.ds(j * _GCHUNK, _GCHUNK)], sem).wait()
    pltpu.sync_copy(rows_v, out_hbm.at[pl.ds(wid * _BPW, _BPW)])


@functools.lru_cache(maxsize=None)
def _sc_gather():
    return pl.kernel(
        _gather_body,
        out_type=jax.ShapeDtypeStruct((_N_TOK, _DIM), jnp.float32),
        mesh=plsc.VectorSubcoreMesh(core_axis_name="c", subcore_axis_name="s",
                                    num_cores=_NC, num_subcores=_NS),
        scratch_types=[
            pltpu.VMEM((_BPW // _GCHUNK, _GCHUNK), jnp.int32),
            pltpu.VMEM((_BPW, _DIM), jnp.float32),
            pltpu.SemaphoreType.DMA,
        ],
        compiler_params=pltpu.CompilerParams(use_tc_tiling_on_sc=False),
    )


def _st_loss_body(zt_ref, zq_ref, out_ref, part_ref):
    zt = zt_ref[...]
    zq = zq_ref[...]
    out_ref[...] = zt + (zq - zt)
    diff = zq - zt
    part_ref[0, 0, 0] = jnp.sum(diff * diff)


def _st_loss(z_flat, zq_flat):
    return pl.pallas_call(
        _st_loss_body,
        grid=(_N_TOK // _TOK_TILE,),
        in_specs=[
            pl.BlockSpec((_TOK_TILE, _DIM), lambda b: (b, 0)),
            pl.BlockSpec((_TOK_TILE, _DIM), lambda b: (b, 0)),
        ],
        out_specs=[
            pl.BlockSpec((_TOK_TILE, _DIM), lambda b: (b, 0)),
            pl.BlockSpec((1, 1, 1), lambda b: (b, 0, 0),
                         memory_space=pltpu.SMEM),
        ],
        out_shape=[
            jax.ShapeDtypeStruct((_N_TOK, _DIM), jnp.float32),
            jax.ShapeDtypeStruct((_N_TOK // _TOK_TILE, 1, 1), jnp.float32),
        ],
    )(z_flat, zq_flat)


def kernel(z, codebook):
    z_p = jnp.transpose(z, (0, 2, 3, 1))
    z_flat = z_p.reshape(_N_TOK, _DIM)
    zsq3 = jnp.sum(z_flat ** 2, axis=1).reshape(_N_TOK // _TOK_TILE, 1, _TOK_TILE)
    idx = _compute_indices(z_flat, zsq3, codebook)
    idx2 = idx.reshape(_N_TOK // _GCHUNK, _GCHUNK)
    zq_flat = _sc_gather()(codebook, idx2)
    zq_st_flat, parts = _st_loss(z_flat, zq_flat)
    m = jnp.sum(parts) / (_N_TOK * _DIM)
    loss = m + 0.5 * m
    zq_st = jnp.transpose(zq_st_flat.reshape(8, 32, 32, _DIM), (0, 3, 1, 2))
    return zq_st, loss


# speedup vs baseline: 1.4166x; 1.4166x over previous
import functools

import jax
import jax.numpy as jnp
from jax import lax
from jax.experimental import pallas as pl
from jax.experimental.pallas import tpu as pltpu
from jax.experimental.pallas import tpu_sc as plsc

_NUM_EMB = 8192
_DIM = 32
_N_TOK = 8192
_TOK_TILE = 1024
_K_TILE = 1024
_N_K_TILES = _NUM_EMB // _K_TILE
_K_CHUNK = 2048
_N_CHUNKS = _NUM_EMB // _K_CHUNK
_TILES_PER_CHUNK = _K_CHUNK // _K_TILE

_NC = 2
_NS = 16
_NW = _NC * _NS
_BPW = _N_TOK // _NW
_GCHUNK = 128


def _argmin_body(zt_ref, zsq_ref, cb_ref, idx_ref):
    zt = zt_ref[...]
    zsq = zsq_ref[...].reshape(_TOK_TILE)
    ztb = zt.astype(jnp.bfloat16)

    def tile_step(k, carry):
        run_val, run_idx = carry
        cbk = cb_ref[pl.ds(k * _K_TILE, _K_TILE), :]
        csq = jnp.sum(cbk * cbk, axis=1)
        mm = lax.dot_general(ztb, cbk, (((1,), (1,)), ((), ())),
                             preferred_element_type=jnp.float32)
        d = (zsq[:, None] + csq[None, :]) - 2.0 * mm
        val = jnp.min(d, axis=1)
        lane = lax.broadcasted_iota(jnp.int32, (_TOK_TILE, _K_TILE), 1) + k * _K_TILE
        idx = jnp.min(jnp.where(d == val[:, None], lane, jnp.int32(2**30)), axis=1)
        better = val < run_val
        return jnp.where(better, val, run_val), jnp.where(better, idx, run_idx)

    def chunk_step(c, carry):
        acc_val, acc_idx = carry
        init = (jnp.full((_TOK_TILE,), jnp.inf, jnp.float32),
                jnp.zeros((_TOK_TILE,), jnp.int32))
        v, i = lax.fori_loop(c * _TILES_PER_CHUNK, (c + 1) * _TILES_PER_CHUNK,
                             tile_step, init)
        acc_r = acc_val.astype(jnp.bfloat16).astype(jnp.float32)
        better = v < acc_r
        return jnp.where(better, v, acc_r), jnp.where(better, i, acc_idx)

    init = (jnp.full((_TOK_TILE,), jnp.inf, jnp.float32),
            jnp.zeros((_TOK_TILE,), jnp.int32))
    _, run_idx = lax.fori_loop(0, _N_CHUNKS, chunk_step, init)
    idx_ref[...] = run_idx.reshape(1, 1, _TOK_TILE)


def _compute_indices(z_flat, zsq3, codebook):
    return pl.pallas_call(
        _argmin_body,
        grid=(_N_TOK // _TOK_TILE,),
        in_specs=[
            pl.BlockSpec((_TOK_TILE, _DIM), lambda b: (b, 0)),
            pl.BlockSpec((1, 1, _TOK_TILE), lambda b: (b, 0, 0)),
            pl.BlockSpec((_NUM_EMB, _DIM), lambda b: (0, 0)),
        ],
        out_specs=pl.BlockSpec((1, 1, _TOK_TILE), lambda b: (b, 0, 0)),
        out_shape=jax.ShapeDtypeStruct((_N_TOK // _TOK_TILE, 1, _TOK_TILE),
                                       jnp.int32),
    )(z_flat, zsq3, codebook)


def _gather_body(cb_hbm, idx_hbm, out_hbm, idx_v, rows_v, sem):
    wid = lax.axis_index("s") * _NC + lax.axis_index("c")
    nrow = _BPW // _GCHUNK
    pltpu.sync_copy(idx_hbm.at[pl.ds(wid * nrow, nrow)], idx_v)
    for j in range(nrow):
        pltpu.async_copy(cb_hbm.at[idx_v.at[j]],
                         rows_v.at[pl.ds(j * _GCHUNK, _GCHUNK)], sem).wait()
    pltpu.sync_copy(rows_v, out_hbm.at[pl.ds(wid * _BPW, _BPW)])


@functools.lru_cache(maxsize=None)
def _sc_gather():
    return pl.kernel(
        _gather_body,
        out_type=jax.ShapeDtypeStruct((_N_TOK, _DIM), jnp.float32),
        mesh=plsc.VectorSubcoreMesh(core_axis_name="c", subcore_axis_name="s",
                                    num_cores=_NC, num_subcores=_NS),
        scratch_types=[
            pltpu.VMEM((_BPW // _GCHUNK, _GCHUNK), jnp.int32),
            pltpu.VMEM((_BPW, _DIM), jnp.float32),
            pltpu.SemaphoreType.DMA,
        ],
        compiler_params=pltpu.CompilerParams(use_tc_tiling_on_sc=False),
    )


def _st_loss_body(zt_ref, zq_ref, out_ref, part_ref):
    zt = zt_ref[...]
    zq = zq_ref[...]
    out_ref[...] = zt + (zq - zt)
    diff = zq - zt
    part_ref[0, 0, 0] = jnp.sum(diff * diff)


def _st_loss(z_flat, zq_flat):
    return pl.pallas_call(
        _st_loss_body,
        grid=(_N_TOK // _TOK_TILE,),
        in_specs=[
            pl.BlockSpec((_TOK_TILE, _DIM), lambda b: (b, 0)),
            pl.BlockSpec((_TOK_TILE, _DIM), lambda b: (b, 0)),
        ],
        out_specs=[
            pl.BlockSpec((_TOK_TILE, _DIM), lambda b: (b, 0)),
            pl.BlockSpec((1, 1, 1), lambda b: (b, 0, 0),
                         memory_space=pltpu.SMEM),
        ],
        out_shape=[
            jax.ShapeDtypeStruct((_N_TOK, _DIM), jnp.float32),
            jax.ShapeDtypeStruct((_N_TOK // _TOK_TILE, 1, 1), jnp.float32),
        ],
    )(z_flat, zq_flat)


def kernel(z, codebook):
    z_p = jnp.transpose(z, (0, 2, 3, 1))
    z_flat = z_p.reshape(_N_TOK, _DIM)
    zsq3 = jnp.sum(z_flat ** 2, axis=1).reshape(_N_TOK // _TOK_TILE, 1, _TOK_TILE)
    idx = _compute_indices(z_flat, zsq3, codebook)
    idx2 = idx.reshape(_N_TOK // _GCHUNK, _GCHUNK)
    zq_flat = _sc_gather()(codebook, idx2)
    zq_st_flat, parts = _st_loss(z_flat, zq_flat)
    m = jnp.sum(parts) / (_N_TOK * _DIM)
    loss = m + 0.5 * m
    zq_st = jnp.transpose(zq_st_flat.reshape(8, 32, 32, _DIM), (0, 3, 1, 2))
    return zq_st, loss


# speedup vs baseline: 1.5702x; 1.1084x over previous
import functools

import jax
import jax.numpy as jnp
from jax import lax
from jax.experimental import pallas as pl
from jax.experimental.pallas import tpu as pltpu
from jax.experimental.pallas import tpu_sc as plsc

_NUM_EMB = 8192
_DIM = 32
_N_TOK = 8192
_TOK_TILE = 1024
_K_TILE = 2048
_N_K_TILES = _NUM_EMB // _K_TILE
_K_CHUNK = 2048
_N_CHUNKS = _NUM_EMB // _K_CHUNK
_TILES_PER_CHUNK = _K_CHUNK // _K_TILE

_NC = 2
_NS = 16
_NW = _NC * _NS
_BPW = _N_TOK // _NW
_GCHUNK = 128


def _argmin_body(zt_ref, zsq_ref, cb_ref, idx_ref):
    zt = zt_ref[...]
    zsq = zsq_ref[...].reshape(_TOK_TILE)
    ztb = zt.astype(jnp.bfloat16)

    def tile_step(k, carry):
        run_val, run_idx = carry
        cbk = cb_ref[pl.ds(k * _K_TILE, _K_TILE), :]
        csq = jnp.sum(cbk * cbk, axis=1)
        mm = lax.dot_general(ztb, cbk, (((1,), (1,)), ((), ())),
                             preferred_element_type=jnp.float32)
        d = (zsq[:, None] + csq[None, :]) - 2.0 * mm
        val = jnp.min(d, axis=1)
        lane = lax.broadcasted_iota(jnp.int32, (_TOK_TILE, _K_TILE), 1) + k * _K_TILE
        idx = jnp.min(jnp.where(d == val[:, None], lane, jnp.int32(2**30)), axis=1)
        better = val < run_val
        return jnp.where(better, val, run_val), jnp.where(better, idx, run_idx)

    def chunk_step(c, carry):
        acc_val, acc_idx = carry
        init = (jnp.full((_TOK_TILE,), jnp.inf, jnp.float32),
                jnp.zeros((_TOK_TILE,), jnp.int32))
        v, i = lax.fori_loop(c * _TILES_PER_CHUNK, (c + 1) * _TILES_PER_CHUNK,
                             tile_step, init)
        acc_r = acc_val.astype(jnp.bfloat16).astype(jnp.float32)
        better = v < acc_r
        return jnp.where(better, v, acc_r), jnp.where(better, i, acc_idx)

    init = (jnp.full((_TOK_TILE,), jnp.inf, jnp.float32),
            jnp.zeros((_TOK_TILE,), jnp.int32))
    _, run_idx = lax.fori_loop(0, _N_CHUNKS, chunk_step, init)
    idx_ref[...] = run_idx.reshape(1, 1, _TOK_TILE)


def _compute_indices(z_flat, zsq3, codebook):
    return pl.pallas_call(
        _argmin_body,
        grid=(_N_TOK // _TOK_TILE,),
        in_specs=[
            pl.BlockSpec((_TOK_TILE, _DIM), lambda b: (b, 0)),
            pl.BlockSpec((1, 1, _TOK_TILE), lambda b: (b, 0, 0)),
            pl.BlockSpec((_NUM_EMB, _DIM), lambda b: (0, 0)),
        ],
        out_specs=pl.BlockSpec((1, 1, _TOK_TILE), lambda b: (b, 0, 0)),
        out_shape=jax.ShapeDtypeStruct((_N_TOK // _TOK_TILE, 1, _TOK_TILE),
                                       jnp.int32),
    )(z_flat, zsq3, codebook)


def _gather_body(cb_hbm, idx_hbm, out_hbm, idx_v, rows_v, sem):
    wid = lax.axis_index("s") * _NC + lax.axis_index("c")
    nrow = _BPW // _GCHUNK
    pltpu.sync_copy(idx_hbm.at[pl.ds(wid * nrow, nrow)], idx_v)
    for j in range(nrow):
        pltpu.async_copy(cb_hbm.at[idx_v.at[j]],
                         rows_v.at[pl.ds(j * _GCHUNK, _GCHUNK)], sem).wait()
    pltpu.sync_copy(rows_v, out_hbm.at[pl.ds(wid * _BPW, _BPW)])


@functools.lru_cache(maxsize=None)
def _sc_gather():
    return pl.kernel(
        _gather_body,
        out_type=jax.ShapeDtypeStruct((_N_TOK, _DIM), jnp.float32),
        mesh=plsc.VectorSubcoreMesh(core_axis_name="c", subcore_axis_name="s",
                                    num_cores=_NC, num_subcores=_NS),
        scratch_types=[
            pltpu.VMEM((_BPW // _GCHUNK, _GCHUNK), jnp.int32),
            pltpu.VMEM((_BPW, _DIM), jnp.float32),
            pltpu.SemaphoreType.DMA,
        ],
        compiler_params=pltpu.CompilerParams(use_tc_tiling_on_sc=False),
    )


def _st_loss_body(zt_ref, zq_ref, out_ref, part_ref):
    zt = zt_ref[...]
    zq = zq_ref[...]
    out_ref[...] = zt + (zq - zt)
    diff = zq - zt
    part_ref[0, 0, 0] = jnp.sum(diff * diff)


def _st_loss(z_flat, zq_flat):
    return pl.pallas_call(
        _st_loss_body,
        grid=(_N_TOK // _TOK_TILE,),
        in_specs=[
            pl.BlockSpec((_TOK_TILE, _DIM), lambda b: (b, 0)),
            pl.BlockSpec((_TOK_TILE, _DIM), lambda b: (b, 0)),
        ],
        out_specs=[
            pl.BlockSpec((_TOK_TILE, _DIM), lambda b: (b, 0)),
            pl.BlockSpec((1, 1, 1), lambda b: (b, 0, 0),
                         memory_space=pltpu.SMEM),
        ],
        out_shape=[
            jax.ShapeDtypeStruct((_N_TOK, _DIM), jnp.float32),
            jax.ShapeDtypeStruct((_N_TOK // _TOK_TILE, 1, 1), jnp.float32),
        ],
    )(z_flat, zq_flat)


def kernel(z, codebook):
    z_p = jnp.transpose(z, (0, 2, 3, 1))
    z_flat = z_p.reshape(_N_TOK, _DIM)
    zsq3 = jnp.sum(z_flat ** 2, axis=1).reshape(_N_TOK // _TOK_TILE, 1, _TOK_TILE)
    idx = _compute_indices(z_flat, zsq3, codebook)
    idx2 = idx.reshape(_N_TOK // _GCHUNK, _GCHUNK)
    zq_flat = _sc_gather()(codebook, idx2)
    zq_st_flat, parts = _st_loss(z_flat, zq_flat)
    m = jnp.sum(parts) / (_N_TOK * _DIM)
    loss = m + 0.5 * m
    zq_st = jnp.transpose(zq_st_flat.reshape(8, 32, 32, _DIM), (0, 3, 1, 2))
    return zq_st, loss


# speedup vs baseline: 1.6549x; 1.0540x over previous
import functools

import jax
import jax.numpy as jnp
from jax import lax
from jax.experimental import pallas as pl
from jax.experimental.pallas import tpu as pltpu
from jax.experimental.pallas import tpu_sc as plsc

_NUM_EMB = 8192
_DIM = 32
_N_TOK = 8192
_TOK_TILE = 2048
_K_TILE = 2048
_N_K_TILES = _NUM_EMB // _K_TILE
_K_CHUNK = 2048
_N_CHUNKS = _NUM_EMB // _K_CHUNK
_TILES_PER_CHUNK = _K_CHUNK // _K_TILE

_NC = 2
_NS = 16
_NW = _NC * _NS
_BPW = _N_TOK // _NW
_GCHUNK = 128


def _argmin_body(zt_ref, zsq_ref, cb_ref, idx_ref):
    zt = zt_ref[...]
    zsq = zsq_ref[...].reshape(_TOK_TILE)
    ztb = zt.astype(jnp.bfloat16)

    def tile_step(k, carry):
        run_val, run_idx = carry
        cbk = cb_ref[pl.ds(k * _K_TILE, _K_TILE), :]
        csq = jnp.sum(cbk * cbk, axis=1)
        mm = lax.dot_general(ztb, cbk, (((1,), (1,)), ((), ())),
                             preferred_element_type=jnp.float32)
        d = (zsq[:, None] + csq[None, :]) - 2.0 * mm
        val = jnp.min(d, axis=1)
        lane = lax.broadcasted_iota(jnp.int32, (_TOK_TILE, _K_TILE), 1) + k * _K_TILE
        idx = jnp.min(jnp.where(d == val[:, None], lane, jnp.int32(2**30)), axis=1)
        better = val < run_val
        return jnp.where(better, val, run_val), jnp.where(better, idx, run_idx)

    def chunk_step(c, carry):
        acc_val, acc_idx = carry
        init = (jnp.full((_TOK_TILE,), jnp.inf, jnp.float32),
                jnp.zeros((_TOK_TILE,), jnp.int32))
        v, i = lax.fori_loop(c * _TILES_PER_CHUNK, (c + 1) * _TILES_PER_CHUNK,
                             tile_step, init)
        acc_r = acc_val.astype(jnp.bfloat16).astype(jnp.float32)
        better = v < acc_r
        return jnp.where(better, v, acc_r), jnp.where(better, i, acc_idx)

    init = (jnp.full((_TOK_TILE,), jnp.inf, jnp.float32),
            jnp.zeros((_TOK_TILE,), jnp.int32))
    _, run_idx = lax.fori_loop(0, _N_CHUNKS, chunk_step, init)
    idx_ref[...] = run_idx.reshape(1, 1, _TOK_TILE)


def _compute_indices(z_flat, zsq3, codebook):
    return pl.pallas_call(
        _argmin_body,
        grid=(_N_TOK // _TOK_TILE,),
        in_specs=[
            pl.BlockSpec((_TOK_TILE, _DIM), lambda b: (b, 0)),
            pl.BlockSpec((1, 1, _TOK_TILE), lambda b: (b, 0, 0)),
            pl.BlockSpec((_NUM_EMB, _DIM), lambda b: (0, 0)),
        ],
        out_specs=pl.BlockSpec((1, 1, _TOK_TILE), lambda b: (b, 0, 0)),
        out_shape=jax.ShapeDtypeStruct((_N_TOK // _TOK_TILE, 1, _TOK_TILE),
                                       jnp.int32),
    )(z_flat, zsq3, codebook)


def _gather_body(cb_hbm, idx_hbm, out_hbm, idx_v, rows_v, sem):
    wid = lax.axis_index("s") * _NC + lax.axis_index("c")
    nrow = _BPW // _GCHUNK
    pltpu.sync_copy(idx_hbm.at[pl.ds(wid * nrow, nrow)], idx_v)
    for j in range(nrow):
        pltpu.async_copy(cb_hbm.at[idx_v.at[j]],
                         rows_v.at[pl.ds(j * _GCHUNK, _GCHUNK)], sem).wait()
    pltpu.sync_copy(rows_v, out_hbm.at[pl.ds(wid * _BPW, _BPW)])


@functools.lru_cache(maxsize=None)
def _sc_gather():
    return pl.kernel(
        _gather_body,
        out_type=jax.ShapeDtypeStruct((_N_TOK, _DIM), jnp.float32),
        mesh=plsc.VectorSubcoreMesh(core_axis_name="c", subcore_axis_name="s",
                                    num_cores=_NC, num_subcores=_NS),
        scratch_types=[
            pltpu.VMEM((_BPW // _GCHUNK, _GCHUNK), jnp.int32),
            pltpu.VMEM((_BPW, _DIM), jnp.float32),
            pltpu.SemaphoreType.DMA,
        ],
        compiler_params=pltpu.CompilerParams(use_tc_tiling_on_sc=False),
    )


def _st_loss_body(zt_ref, zq_ref, out_ref, part_ref):
    zt = zt_ref[...]
    zq = zq_ref[...]
    out_ref[...] = zt + (zq - zt)
    diff = zq - zt
    part_ref[0, 0, 0] = jnp.sum(diff * diff)


def _st_loss(z_flat, zq_flat):
    return pl.pallas_call(
        _st_loss_body,
        grid=(_N_TOK // _TOK_TILE,),
        in_specs=[
            pl.BlockSpec((_TOK_TILE, _DIM), lambda b: (b, 0)),
            pl.BlockSpec((_TOK_TILE, _DIM), lambda b: (b, 0)),
        ],
        out_specs=[
            pl.BlockSpec((_TOK_TILE, _DIM), lambda b: (b, 0)),
            pl.BlockSpec((1, 1, 1), lambda b: (b, 0, 0),
                         memory_space=pltpu.SMEM),
        ],
        out_shape=[
            jax.ShapeDtypeStruct((_N_TOK, _DIM), jnp.float32),
            jax.ShapeDtypeStruct((_N_TOK // _TOK_TILE, 1, 1), jnp.float32),
        ],
    )(z_flat, zq_flat)


def kernel(z, codebook):
    z_p = jnp.transpose(z, (0, 2, 3, 1))
    z_flat = z_p.reshape(_N_TOK, _DIM)
    zsq3 = jnp.sum(z_flat ** 2, axis=1).reshape(_N_TOK // _TOK_TILE, 1, _TOK_TILE)
    idx = _compute_indices(z_flat, zsq3, codebook)
    idx2 = idx.reshape(_N_TOK // _GCHUNK, _GCHUNK)
    zq_flat = _sc_gather()(codebook, idx2)
    zq_st_flat, parts = _st_loss(z_flat, zq_flat)
    m = jnp.sum(parts) / (_N_TOK * _DIM)
    loss = m + 0.5 * m
    zq_st = jnp.transpose(zq_st_flat.reshape(8, 32, 32, _DIM), (0, 3, 1, 2))
    return zq_st, loss
